# Initial kernel scaffold; baseline (speedup 1.0000x reference)
import jax
import jax.numpy as jnp
from jax.experimental import pallas as pl


def kernel(x, edge_index, batch, conv_W0, conv_b0, conv_W1, conv_b1, conv_W2, conv_b2, conv_W3, conv_b3, ro_W0, ro_b0, ro_W1, ro_b1):
    raise NotImplementedError("write your pallas kernel here")


# speedup vs baseline: 5.7932x; 5.7932x over previous
import functools

import jax
import jax.numpy as jnp
from jax import lax
from jax.experimental import pallas as pl
from jax.experimental.pallas import tpu as pltpu
from jax.experimental.pallas import tpu_sc as plsc

N = 50000
E = 800000
C = 128
KORD = 6
G = 64

NC, NS, LANES = 2, 16, 16
NW = NC * NS

BLK = 6272
NPAD = 8 * BLK
DUMV = BLK
ACC_ROWS = BLK + 16
CH = 256
EPAD = 1536
BN = 512
NBLK = NPAD // BN
ZR = 98

_mesh = plsc.VectorSubcoreMesh(core_axis_name="c", subcore_axis_name="s")

NEG_BIG = -3.0e38


def _mask_chunk(lidx, e0, lo, hi, nvec):
    iota = lax.iota(jnp.int32, 16)
    for w in range(nvec):
        pos = iota + (e0 + 16 * w)
        lv = lidx[pl.ds(16 * w, 16)]
        keep = jnp.logical_and(pos >= lo, pos < hi)
        lidx[pl.ds(16 * w, 16)] = jnp.where(keep, lv, DUMV)


def _extract9(w, b):
    lo = w[0]
    hi = w[1]
    for i in range(1, 8):
        sel = b >= i
        lo = jnp.where(sel, w[i], lo)
        hi = jnp.where(sel, w[i + 1], hi)
    return lo, hi


@functools.partial(
    pl.kernel,
    out_type=jax.ShapeDtypeStruct((NPAD, C), jnp.float32),
    mesh=_mesh,
    scratch_types=[
        pltpu.VMEM((16,), jnp.int32),
        pltpu.VMEM((CH,), jnp.int32),
        pltpu.VMEM((CH,), jnp.int32),
        pltpu.VMEM((CH,), jnp.int32),
        pltpu.VMEM((CH,), jnp.int32),
        pltpu.VMEM((CH, C), jnp.float32),
        pltpu.VMEM((CH, C), jnp.float32),
        pltpu.VMEM((ZR, C), jnp.float32),
        pltpu.VMEM_SHARED((ACC_ROWS, C), jnp.float32),
        pltpu.SemaphoreType.DMA,
        pltpu.SemaphoreType.DMA,
        pltpu.SemaphoreType.DMA,
        pltpu.SemaphoreType.DMA,
    ],
)
def _spmm_kernel(hs, rowe, loce, meta, out,
                 mbuf, ridxa, ridxb, lidxa, lidxb, stga, stgb, zbuf, acc,
                 sga, sgb, ssa, ssb):
    c = lax.axis_index("c")
    s = lax.axis_index("s")
    pltpu.sync_copy(meta.at[pl.ds(0, 16)], mbuf)
    w = mbuf[pl.ds(0, 16)]
    for u in range(ZR):
        for q in range(C // 16):
            zbuf[u, pl.ds(16 * q, 16)] = jnp.zeros((16,), jnp.float32)
    for u in range(4):
        b = 4 * c + u
        lo, hi = _extract9(w, b)
        base_node = b * BLK
        row0 = s * (BLK // NS)
        for z in range(BLK // NS // ZR):
            pltpu.sync_copy(zbuf, acc.at[pl.ds(row0 + z * ZR, ZR)])
        plsc.subcore_barrier()

        span = hi - lo
        share = lax.shift_left(
            lax.shift_right_logical(span + (NS * CH - 1), 12), 8)
        a = lo + s * share
        bnd = jnp.minimum(a + share, hi)
        a8 = jnp.bitwise_and(a, jnp.int32(-8))
        npair = lax.shift_right_logical(
            jnp.maximum(bnd - a8, 0) + (2 * CH - 1), 9)

        def pair_body(j, carry):
            e0 = pl.multiple_of(a8 + j * (2 * CH), 8)
            e1 = pl.multiple_of(e0 + CH, 8)
            pltpu.sync_copy(rowe.at[pl.ds(e0, CH)], ridxa)
            pltpu.sync_copy(loce.at[pl.ds(e0, CH)], lidxa)
            pltpu.sync_copy(rowe.at[pl.ds(e1, CH)], ridxb)
            pltpu.sync_copy(loce.at[pl.ds(e1, CH)], lidxb)
            _mask_chunk(lidxa, e0, a, bnd, CH // 16)
            _mask_chunk(lidxb, e1, a, bnd, CH // 16)
            cpa = pltpu.async_copy(hs.at[ridxa], stga, sga)
            cpb = pltpu.async_copy(hs.at[ridxb], stgb, sgb)
            cpa.wait()
            wa = pltpu.async_copy(stga, acc.at[lidxa], ssa, add=True)
            cpb.wait()
            wb = pltpu.async_copy(stgb, acc.at[lidxb], ssb, add=True)
            wa.wait()
            wb.wait()
            return carry

        lax.fori_loop(0, npair, pair_body, jnp.int32(0))
        plsc.subcore_barrier()
        nwb = BLK // NS
        pltpu.sync_copy(acc.at[pl.ds(row0, nwb)],
                        out.at[pl.ds(base_node + row0, nwb)])
        plsc.subcore_barrier()


@functools.partial(
    pl.kernel,
    out_type=jax.ShapeDtypeStruct((NPAD, C), jnp.float32),
    mesh=_mesh,
    scratch_types=[
        pltpu.VMEM((16,), jnp.int32),
        pltpu.VMEM((CH,), jnp.int32),
        pltpu.VMEM((CH,), jnp.int32),
        pltpu.VMEM((CH, C), jnp.float32),
        pltpu.VMEM((ZR, C), jnp.float32),
        pltpu.VMEM_SHARED((ACC_ROWS, C), jnp.float32),
        pltpu.SemaphoreType.DMA,
        pltpu.SemaphoreType.DMA,
    ],
)
def _deg_kernel(locr, meta, out, mbuf, lidxa, lidxb, ones, zbuf, acc, ssa, ssb):
    c = lax.axis_index("c")
    s = lax.axis_index("s")
    pltpu.sync_copy(meta.at[pl.ds(0, 16)], mbuf)
    w = mbuf[pl.ds(0, 16)]
    for u in range(ZR):
        for q in range(C // 16):
            zbuf[u, pl.ds(16 * q, 16)] = jnp.zeros((16,), jnp.float32)
    for u in range(CH):
        for q in range(C // 16):
            ones[u, pl.ds(16 * q, 16)] = jnp.full((16,), 1.0, jnp.float32)
    for u in range(4):
        b = 4 * c + u
        lo, hi = _extract9(w, b)
        base_node = b * BLK
        row0 = s * (BLK // NS)
        for z in range(BLK // NS // ZR):
            pltpu.sync_copy(zbuf, acc.at[pl.ds(row0 + z * ZR, ZR)])
        plsc.subcore_barrier()

        span = hi - lo
        share = lax.shift_left(
            lax.shift_right_logical(span + (NS * CH - 1), 12), 8)
        a = lo + s * share
        bnd = jnp.minimum(a + share, hi)
        a8 = jnp.bitwise_and(a, jnp.int32(-8))
        npair = lax.shift_right_logical(
            jnp.maximum(bnd - a8, 0) + (2 * CH - 1), 9)

        def pair_body(j, carry):
            e0 = pl.multiple_of(a8 + j * (2 * CH), 8)
            e1 = pl.multiple_of(e0 + CH, 8)
            pltpu.sync_copy(locr.at[pl.ds(e0, CH)], lidxa)
            pltpu.sync_copy(locr.at[pl.ds(e1, CH)], lidxb)
            _mask_chunk(lidxa, e0, a, bnd, CH // 16)
            _mask_chunk(lidxb, e1, a, bnd, CH // 16)
            wa = pltpu.async_copy(ones, acc.at[lidxa], ssa, add=True)
            wb = pltpu.async_copy(ones, acc.at[lidxb], ssb, add=True)
            wa.wait()
            wb.wait()
            return carry

        lax.fori_loop(0, npair, pair_body, jnp.int32(0))
        plsc.subcore_barrier()
        nwb = BLK // NS
        pltpu.sync_copy(acc.at[pl.ds(row0, nwb)],
                        out.at[pl.ds(base_node + row0, nwb)])
        plsc.subcore_barrier()


def _rowspec():
    return pl.BlockSpec((BN, C), lambda i: (i, 0))


def _wspec():
    return pl.BlockSpec((C, C), lambda i: (0, 0))


def _bspec():
    return pl.BlockSpec((1, C), lambda i: (0, 0))


def _accspec():
    return pl.BlockSpec((8, C), lambda i: (0, 0))


def _prep_body(deg_ref, x_ref, dis_ref, hs_ref):
    d = deg_ref[...]
    safe = jnp.where(d > 0, d, 1.0)
    dis = jnp.where(d > 0, 1.0 / jnp.sqrt(safe), 0.0)
    dis_ref[...] = dis
    hs_ref[...] = dis * x_ref[...]


def _prep(deg128, x_pad):
    return pl.pallas_call(
        _prep_body,
        grid=(NBLK,),
        in_specs=[_rowspec(), _rowspec()],
        out_specs=[_rowspec(), _rowspec()],
        out_shape=[jax.ShapeDtypeStruct((NPAD, C), jnp.float32),
                   jax.ShapeDtypeStruct((NPAD, C), jnp.float32)],
    )(deg128, x_pad)


def _step1_body(h_ref, p_ref, dis_ref, w0_ref, w1_ref,
                acc_ref, tx_ref, hs_ref):
    dis = dis_ref[...]
    t1 = -dis * p_ref[...]
    acc = jnp.dot(h_ref[...], w0_ref[...], preferred_element_type=jnp.float32)
    acc = acc + jnp.dot(t1, w1_ref[...], preferred_element_type=jnp.float32)
    acc_ref[...] = acc
    tx_ref[...] = t1
    hs_ref[...] = dis * t1


def _step1(h, p, dis128, w0, w1):
    return pl.pallas_call(
        _step1_body,
        grid=(NBLK,),
        in_specs=[_rowspec(), _rowspec(), _rowspec(), _wspec(), _wspec()],
        out_specs=[_rowspec(), _rowspec(), _rowspec()],
        out_shape=[jax.ShapeDtypeStruct((NPAD, C), jnp.float32)] * 3,
    )(h, p, dis128, w0, w1)


def _stepm_body(p_ref, txm2_ref, acc_ref_in, dis_ref, w_ref,
                acc_ref, tx_ref, hs_ref):
    dis = dis_ref[...]
    t = -2.0 * dis * p_ref[...] - txm2_ref[...]
    acc_ref[...] = acc_ref_in[...] + jnp.dot(
        t, w_ref[...], preferred_element_type=jnp.float32)
    tx_ref[...] = t
    hs_ref[...] = dis * t


def _stepm(p, txm2, acc, dis128, wk):
    return pl.pallas_call(
        _stepm_body,
        grid=(NBLK,),
        in_specs=[_rowspec(), _rowspec(), _rowspec(), _rowspec(), _wspec()],
        out_specs=[_rowspec(), _rowspec(), _rowspec()],
        out_shape=[jax.ShapeDtypeStruct((NPAD, C), jnp.float32)] * 3,
    )(p, txm2, acc, dis128, wk)


def _stepl_body(p_ref, txm2_ref, acc_ref_in, dis_ref, w_ref, b_ref,
                h_ref, hs_ref):
    dis = dis_ref[...]
    t = -2.0 * dis * p_ref[...] - txm2_ref[...]
    acc = acc_ref_in[...] + jnp.dot(
        t, w_ref[...], preferred_element_type=jnp.float32)
    h = jnp.maximum(acc + b_ref[...], 0.0)
    h_ref[...] = h
    hs_ref[...] = dis * h


def _stepl(p, txm2, acc, dis128, wk, bias):
    return pl.pallas_call(
        _stepl_body,
        grid=(NBLK,),
        in_specs=[_rowspec(), _rowspec(), _rowspec(), _rowspec(), _wspec(),
                  _bspec()],
        out_specs=[_rowspec(), _rowspec()],
        out_shape=[jax.ShapeDtypeStruct((NPAD, C), jnp.float32)] * 2,
    )(p, txm2, acc, dis128, wk, bias)


def _readout_body(h_ref, w0_ref, b0_ref, w1_ref, b1_ref, oh_ref,
                  s_ref, m_ref):
    i = pl.program_id(0)
    z = jnp.maximum(jnp.dot(h_ref[...], w0_ref[...],
                            preferred_element_type=jnp.float32) + b0_ref[...],
                    0.0)
    sv = jnp.maximum(jnp.dot(z, w1_ref[...],
                             preferred_element_type=jnp.float32) + b1_ref[...],
                     0.0)
    s_ref[...] = sv
    oh = oh_ref[...]
    masked = jnp.where(oh > 0.5, sv, NEG_BIG)
    mx = jnp.max(masked, axis=0, keepdims=True)

    @pl.when(i == 0)
    def _():
        m_ref[...] = jnp.full((8, C), NEG_BIG, jnp.float32)

    m_ref[0:1, :] = jnp.maximum(m_ref[0:1, :], mx)


def _readout(h, w0, b0, w1b, b1b, onehot):
    return pl.pallas_call(
        _readout_body,
        grid=(NBLK,),
        in_specs=[_rowspec(), _wspec(), _bspec(), _wspec(), _bspec(),
                  _rowspec()],
        out_specs=[_rowspec(), _accspec()],
        out_shape=[jax.ShapeDtypeStruct((NPAD, C), jnp.float32),
                   jax.ShapeDtypeStruct((8, C), jnp.float32)],
    )(h, w0, b0, w1b, b1b, onehot)


def _colbcast(vec_row):
    r = lax.broadcasted_iota(jnp.int32, (C, C), 0)
    q = lax.broadcasted_iota(jnp.int32, (C, C), 1)
    eye = jnp.where(r == q, 1.0, 0.0).astype(jnp.float32)
    colv = lax.dot_general(eye, vec_row, (((1,), (1,)), ((), ())),
                           preferred_element_type=jnp.float32)
    return jnp.broadcast_to(colv, (C, C))


def _expsum_body(s_ref, oh_ref, m_ref, ex_ref, ss_ref):
    i = pl.program_id(0)
    mrow = m_ref[0:1, :]
    mrow = jnp.where(mrow > NEG_BIG, mrow, 0.0)
    mb = jnp.dot(oh_ref[...], _colbcast(mrow),
                 preferred_element_type=jnp.float32)
    ex = jnp.exp(s_ref[...] - mb)
    ex_ref[...] = ex
    contrib = jnp.sum(oh_ref[...] * ex, axis=0, keepdims=True)

    @pl.when(i == 0)
    def _():
        ss_ref[...] = jnp.zeros((8, C), jnp.float32)

    ss_ref[0:1, :] = ss_ref[0:1, :] + contrib


def _expsum(s128, onehot, m128):
    return pl.pallas_call(
        _expsum_body,
        grid=(NBLK,),
        in_specs=[_rowspec(), _rowspec(), _accspec()],
        out_specs=[_rowspec(), _accspec()],
        out_shape=[jax.ShapeDtypeStruct((NPAD, C), jnp.float32),
                   jax.ShapeDtypeStruct((8, C), jnp.float32)],
    )(s128, onehot, m128)


def _norm_body(ex_ref, oh_ref, ss_ref, o_ref):
    ssrow = ss_ref[0:1, :]
    ssrow = jnp.where(ssrow > 0, ssrow, 1.0)
    ssb = jnp.dot(oh_ref[...], _colbcast(ssrow),
                  preferred_element_type=jnp.float32)
    ssb = jnp.where(ssb > 0, ssb, 1.0)
    o_ref[...] = ex_ref[...] / ssb


def _norm(ex128, onehot, ss128):
    return pl.pallas_call(
        _norm_body,
        grid=(NBLK,),
        in_specs=[_rowspec(), _rowspec(), _accspec()],
        out_specs=_rowspec(),
        out_shape=jax.ShapeDtypeStruct((NPAD, C), jnp.float32),
    )(ex128, onehot, ss128)


def kernel(x, edge_index, batch, conv_W0, conv_b0, conv_W1, conv_b1,
           conv_W2, conv_b2, conv_W3, conv_b3, ro_W0, ro_b0, ro_W1, ro_b1):
    row = edge_index[0]
    col = edge_index[1]

    blocks = jnp.arange(1, 8, dtype=jnp.int32) * BLK
    order_c = jnp.argsort(col)
    colc = col[order_c]
    rowe = jnp.concatenate([row[order_c],
                            jnp.zeros((EPAD,), jnp.int32)])
    loce = jnp.concatenate([colc - (colc // BLK) * BLK,
                            jnp.full((EPAD,), DUMV, jnp.int32)])
    co = jnp.searchsorted(colc, blocks).astype(jnp.int32)

    order_r = jnp.argsort(row)
    rowr = row[order_r]
    locr = jnp.concatenate([rowr - (rowr // BLK) * BLK,
                            jnp.full((EPAD,), DUMV, jnp.int32)])
    ro = jnp.searchsorted(rowr, blocks).astype(jnp.int32)

    zero = jnp.zeros((1,), jnp.int32)
    evec = jnp.full((1,), E, jnp.int32)
    meta_c = jnp.concatenate([zero, co, evec,
                              jnp.zeros((6,), jnp.int32)])
    meta_r = jnp.concatenate([zero, ro, evec,
                              jnp.zeros((6,), jnp.int32)])

    x_pad = jnp.pad(x, ((0, NPAD - N), (0, C - x.shape[1])))
    w0p = jnp.pad(conv_W0, ((0, 0), (0, C - conv_W0.shape[1]), (0, 0)))
    onehot = jnp.pad((batch[:, None] == jnp.arange(G, dtype=batch.dtype)
                      ).astype(jnp.float32),
                     ((0, NPAD - N), (0, C - G)))
    rob0 = ro_b0.reshape(1, C)
    w1b = jnp.broadcast_to(ro_W1, (C, C))
    rob1 = jnp.broadcast_to(ro_b1.reshape(1, 1), (1, C))
    biases = (conv_b0.reshape(1, C), conv_b1.reshape(1, C),
              conv_b2.reshape(1, C), conv_b3.reshape(1, C))
    weights = (w0p, conv_W1, conv_W2, conv_W3)

    deg128 = _deg_kernel(locr, meta_r)
    dis128, hs = _prep(deg128, x_pad)

    h = x_pad
    for layer in range(4):
        wk = weights[layer]
        p1 = _spmm_kernel(hs, rowe, loce, meta_c)
        acc, txm1, hs1 = _step1(h, p1, dis128, wk[0], wk[1])
        txm2 = h
        hs_cur = hs1
        for k in range(2, KORD):
            pk = _spmm_kernel(hs_cur, rowe, loce, meta_c)
            if k < KORD - 1:
                acc, tx, hs_cur = _stepm(pk, txm2, acc, dis128, wk[k])
                txm2, txm1 = txm1, tx
            else:
                h, hs = _stepl(pk, txm2, acc, dis128, wk[k], biases[layer])

    s128, m128 = _readout(h, ro_W0, rob0, w1b, rob1, onehot)
    ex128, ss128 = _expsum(s128, onehot, m128)
    out128 = _norm(ex128, onehot, ss128)
    return out128[:N, 0]


# speedup vs baseline: 6.5470x; 1.1301x over previous
import functools

import jax
import jax.numpy as jnp
from jax import lax
from jax.experimental import pallas as pl
from jax.experimental.pallas import tpu as pltpu
from jax.experimental.pallas import tpu_sc as plsc

N = 50000
E = 800000
C = 128
KORD = 6
G = 64

NC, NS, LANES = 2, 16, 16
NW = NC * NS

BLK = 6272
NPAD = 8 * BLK
DUMV = BLK
ACC_ROWS = BLK + 16
CH = 256
EPAD = 1536
BN = 512
NBLK = NPAD // BN
ZR = 98

_mesh = plsc.VectorSubcoreMesh(core_axis_name="c", subcore_axis_name="s")

NEG_BIG = -3.0e38


def _mask_chunk(lidx, e0, lo, hi, nvec):
    iota = lax.iota(jnp.int32, 16)
    for w in range(nvec):
        pos = iota + (e0 + 16 * w)
        lv = lidx[pl.ds(16 * w, 16)]
        keep = jnp.logical_and(pos >= lo, pos < hi)
        lidx[pl.ds(16 * w, 16)] = jnp.where(keep, lv, DUMV)


def _extract9(w, b):
    lo = w[0]
    hi = w[1]
    for i in range(1, 8):
        sel = b >= i
        lo = jnp.where(sel, w[i], lo)
        hi = jnp.where(sel, w[i + 1], hi)
    return lo, hi


@functools.partial(
    pl.kernel,
    out_type=jax.ShapeDtypeStruct((NPAD, C), jnp.float32),
    mesh=_mesh,
    scratch_types=[
        pltpu.VMEM((16,), jnp.int32),
        pltpu.VMEM((CH,), jnp.int32),
        pltpu.VMEM((CH,), jnp.int32),
        pltpu.VMEM((CH,), jnp.int32),
        pltpu.VMEM((CH,), jnp.int32),
        pltpu.VMEM((CH, C), jnp.float32),
        pltpu.VMEM((CH, C), jnp.float32),
        pltpu.VMEM((ZR, C), jnp.float32),
        pltpu.VMEM_SHARED((ACC_ROWS, C), jnp.float32),
        pltpu.SemaphoreType.DMA,
        pltpu.SemaphoreType.DMA,
        pltpu.SemaphoreType.DMA,
        pltpu.SemaphoreType.DMA,
        pltpu.SemaphoreType.DMA,
        pltpu.SemaphoreType.DMA,
        pltpu.SemaphoreType.DMA,
        pltpu.SemaphoreType.DMA,
    ],
)
def _spmm_kernel(hs, rowe, loce, meta, out,
                 mbuf, ridxa, ridxb, lidxa, lidxb, stga, stgb, zbuf, acc,
                 sga, sgb, ssa, ssb, sia, sja, sib, sjb):
    c = lax.axis_index("c")
    s = lax.axis_index("s")
    pltpu.sync_copy(meta.at[pl.ds(0, 16)], mbuf)
    w = mbuf[pl.ds(0, 16)]
    for u in range(ZR):
        for q in range(C // 16):
            zbuf[u, pl.ds(16 * q, 16)] = jnp.zeros((16,), jnp.float32)
    for u in range(4):
        b = 4 * c + u
        lo, hi = _extract9(w, b)
        base_node = b * BLK
        row0 = s * (BLK // NS)
        for z in range(BLK // NS // ZR):
            pltpu.sync_copy(zbuf, acc.at[pl.ds(row0 + z * ZR, ZR)])
        plsc.subcore_barrier()

        span = hi - lo
        share = lax.shift_left(
            lax.shift_right_logical(span + (NS * CH - 1), 12), 8)
        a = lo + s * share
        bnd = jnp.minimum(a + share, hi)
        a8 = jnp.bitwise_and(a, jnp.int32(-8))
        npair = lax.shift_right_logical(
            jnp.maximum(bnd - a8, 0) + (2 * CH - 1), 9)

        def pair_body(j, carry):
            e0 = pl.multiple_of(a8 + j * (2 * CH), 8)
            e1 = pl.multiple_of(e0 + CH, 8)

            @pl.when(j > 0)
            def _():
                pltpu.make_async_copy(stga, acc.at[lidxa], ssa).wait()
                pltpu.make_async_copy(stgb, acc.at[lidxb], ssb).wait()

            ira = pltpu.async_copy(rowe.at[pl.ds(e0, CH)], ridxa, sia)
            ila = pltpu.async_copy(loce.at[pl.ds(e0, CH)], lidxa, sja)
            irb = pltpu.async_copy(rowe.at[pl.ds(e1, CH)], ridxb, sib)
            ilb = pltpu.async_copy(loce.at[pl.ds(e1, CH)], lidxb, sjb)
            ira.wait()
            ila.wait()
            _mask_chunk(lidxa, e0, a, bnd, CH // 16)
            cpa = pltpu.async_copy(hs.at[ridxa], stga, sga)
            irb.wait()
            ilb.wait()
            _mask_chunk(lidxb, e1, a, bnd, CH // 16)
            cpb = pltpu.async_copy(hs.at[ridxb], stgb, sgb)
            cpa.wait()
            pltpu.async_copy(stga, acc.at[lidxa], ssa, add=True)
            cpb.wait()
            pltpu.async_copy(stgb, acc.at[lidxb], ssb, add=True)
            return carry

        lax.fori_loop(0, npair, pair_body, jnp.int32(0))

        @pl.when(npair > 0)
        def _():
            pltpu.make_async_copy(stga, acc.at[lidxa], ssa).wait()
            pltpu.make_async_copy(stgb, acc.at[lidxb], ssb).wait()
        plsc.subcore_barrier()
        nwb = BLK // NS
        pltpu.sync_copy(acc.at[pl.ds(row0, nwb)],
                        out.at[pl.ds(base_node + row0, nwb)])
        plsc.subcore_barrier()


@functools.partial(
    pl.kernel,
    out_type=jax.ShapeDtypeStruct((NPAD, C), jnp.float32),
    mesh=_mesh,
    scratch_types=[
        pltpu.VMEM((16,), jnp.int32),
        pltpu.VMEM((CH,), jnp.int32),
        pltpu.VMEM((CH,), jnp.int32),
        pltpu.VMEM((CH, C), jnp.float32),
        pltpu.VMEM((ZR, C), jnp.float32),
        pltpu.VMEM_SHARED((ACC_ROWS, C), jnp.float32),
        pltpu.SemaphoreType.DMA,
        pltpu.SemaphoreType.DMA,
        pltpu.SemaphoreType.DMA,
        pltpu.SemaphoreType.DMA,
    ],
)
def _deg_kernel(locr, meta, out, mbuf, lidxa, lidxb, ones, zbuf, acc,
                ssa, ssb, sia, sib):
    c = lax.axis_index("c")
    s = lax.axis_index("s")
    pltpu.sync_copy(meta.at[pl.ds(0, 16)], mbuf)
    w = mbuf[pl.ds(0, 16)]
    for u in range(ZR):
        for q in range(C // 16):
            zbuf[u, pl.ds(16 * q, 16)] = jnp.zeros((16,), jnp.float32)
    for u in range(CH):
        for q in range(C // 16):
            ones[u, pl.ds(16 * q, 16)] = jnp.full((16,), 1.0, jnp.float32)
    for u in range(4):
        b = 4 * c + u
        lo, hi = _extract9(w, b)
        base_node = b * BLK
        row0 = s * (BLK // NS)
        for z in range(BLK // NS // ZR):
            pltpu.sync_copy(zbuf, acc.at[pl.ds(row0 + z * ZR, ZR)])
        plsc.subcore_barrier()

        span = hi - lo
        share = lax.shift_left(
            lax.shift_right_logical(span + (NS * CH - 1), 12), 8)
        a = lo + s * share
        bnd = jnp.minimum(a + share, hi)
        a8 = jnp.bitwise_and(a, jnp.int32(-8))
        npair = lax.shift_right_logical(
            jnp.maximum(bnd - a8, 0) + (2 * CH - 1), 9)

        def pair_body(j, carry):
            e0 = pl.multiple_of(a8 + j * (2 * CH), 8)
            e1 = pl.multiple_of(e0 + CH, 8)

            @pl.when(j > 0)
            def _():
                pltpu.make_async_copy(ones, acc.at[lidxa], ssa).wait()
                pltpu.make_async_copy(ones, acc.at[lidxb], ssb).wait()

            ila = pltpu.async_copy(locr.at[pl.ds(e0, CH)], lidxa, sia)
            ilb = pltpu.async_copy(locr.at[pl.ds(e1, CH)], lidxb, sib)
            ila.wait()
            _mask_chunk(lidxa, e0, a, bnd, CH // 16)
            pltpu.async_copy(ones, acc.at[lidxa], ssa, add=True)
            ilb.wait()
            _mask_chunk(lidxb, e1, a, bnd, CH // 16)
            pltpu.async_copy(ones, acc.at[lidxb], ssb, add=True)
            return carry

        lax.fori_loop(0, npair, pair_body, jnp.int32(0))

        @pl.when(npair > 0)
        def _():
            pltpu.make_async_copy(ones, acc.at[lidxa], ssa).wait()
            pltpu.make_async_copy(ones, acc.at[lidxb], ssb).wait()
        plsc.subcore_barrier()
        nwb = BLK // NS
        pltpu.sync_copy(acc.at[pl.ds(row0, nwb)],
                        out.at[pl.ds(base_node + row0, nwb)])
        plsc.subcore_barrier()


def _rowspec():
    return pl.BlockSpec((BN, C), lambda i: (i, 0))


def _wspec():
    return pl.BlockSpec((C, C), lambda i: (0, 0))


def _bspec():
    return pl.BlockSpec((1, C), lambda i: (0, 0))


def _accspec():
    return pl.BlockSpec((8, C), lambda i: (0, 0))


def _prep_body(deg_ref, x_ref, dis_ref, hs_ref):
    d = deg_ref[...]
    safe = jnp.where(d > 0, d, 1.0)
    dis = jnp.where(d > 0, 1.0 / jnp.sqrt(safe), 0.0)
    dis_ref[...] = dis
    hs_ref[...] = dis * x_ref[...]


def _prep(deg128, x_pad):
    return pl.pallas_call(
        _prep_body,
        grid=(NBLK,),
        in_specs=[_rowspec(), _rowspec()],
        out_specs=[_rowspec(), _rowspec()],
        out_shape=[jax.ShapeDtypeStruct((NPAD, C), jnp.float32),
                   jax.ShapeDtypeStruct((NPAD, C), jnp.float32)],
    )(deg128, x_pad)


def _step1_body(h_ref, p_ref, dis_ref, w0_ref, w1_ref,
                acc_ref, tx_ref, hs_ref):
    dis = dis_ref[...]
    t1 = -dis * p_ref[...]
    acc = jnp.dot(h_ref[...], w0_ref[...], preferred_element_type=jnp.float32)
    acc = acc + jnp.dot(t1, w1_ref[...], preferred_element_type=jnp.float32)
    acc_ref[...] = acc
    tx_ref[...] = t1
    hs_ref[...] = dis * t1


def _step1(h, p, dis128, w0, w1):
    return pl.pallas_call(
        _step1_body,
        grid=(NBLK,),
        in_specs=[_rowspec(), _rowspec(), _rowspec(), _wspec(), _wspec()],
        out_specs=[_rowspec(), _rowspec(), _rowspec()],
        out_shape=[jax.ShapeDtypeStruct((NPAD, C), jnp.float32)] * 3,
    )(h, p, dis128, w0, w1)


def _stepm_body(p_ref, txm2_ref, acc_ref_in, dis_ref, w_ref,
                acc_ref, tx_ref, hs_ref):
    dis = dis_ref[...]
    t = -2.0 * dis * p_ref[...] - txm2_ref[...]
    acc_ref[...] = acc_ref_in[...] + jnp.dot(
        t, w_ref[...], preferred_element_type=jnp.float32)
    tx_ref[...] = t
    hs_ref[...] = dis * t


def _stepm(p, txm2, acc, dis128, wk):
    return pl.pallas_call(
        _stepm_body,
        grid=(NBLK,),
        in_specs=[_rowspec(), _rowspec(), _rowspec(), _rowspec(), _wspec()],
        out_specs=[_rowspec(), _rowspec(), _rowspec()],
        out_shape=[jax.ShapeDtypeStruct((NPAD, C), jnp.float32)] * 3,
    )(p, txm2, acc, dis128, wk)


def _stepl_body(p_ref, txm2_ref, acc_ref_in, dis_ref, w_ref, b_ref,
                h_ref, hs_ref):
    dis = dis_ref[...]
    t = -2.0 * dis * p_ref[...] - txm2_ref[...]
    acc = acc_ref_in[...] + jnp.dot(
        t, w_ref[...], preferred_element_type=jnp.float32)
    h = jnp.maximum(acc + b_ref[...], 0.0)
    h_ref[...] = h
    hs_ref[...] = dis * h


def _stepl(p, txm2, acc, dis128, wk, bias):
    return pl.pallas_call(
        _stepl_body,
        grid=(NBLK,),
        in_specs=[_rowspec(), _rowspec(), _rowspec(), _rowspec(), _wspec(),
                  _bspec()],
        out_specs=[_rowspec(), _rowspec()],
        out_shape=[jax.ShapeDtypeStruct((NPAD, C), jnp.float32)] * 2,
    )(p, txm2, acc, dis128, wk, bias)


def _readout_body(h_ref, w0_ref, b0_ref, w1_ref, b1_ref, oh_ref,
                  s_ref, m_ref):
    i = pl.program_id(0)
    z = jnp.maximum(jnp.dot(h_ref[...], w0_ref[...],
                            preferred_element_type=jnp.float32) + b0_ref[...],
                    0.0)
    sv = jnp.maximum(jnp.dot(z, w1_ref[...],
                             preferred_element_type=jnp.float32) + b1_ref[...],
                     0.0)
    s_ref[...] = sv
    oh = oh_ref[...]
    masked = jnp.where(oh > 0.5, sv, NEG_BIG)
    mx = jnp.max(masked, axis=0, keepdims=True)

    @pl.when(i == 0)
    def _():
        m_ref[...] = jnp.full((8, C), NEG_BIG, jnp.float32)

    m_ref[0:1, :] = jnp.maximum(m_ref[0:1, :], mx)


def _readout(h, w0, b0, w1b, b1b, onehot):
    return pl.pallas_call(
        _readout_body,
        grid=(NBLK,),
        in_specs=[_rowspec(), _wspec(), _bspec(), _wspec(), _bspec(),
                  _rowspec()],
        out_specs=[_rowspec(), _accspec()],
        out_shape=[jax.ShapeDtypeStruct((NPAD, C), jnp.float32),
                   jax.ShapeDtypeStruct((8, C), jnp.float32)],
    )(h, w0, b0, w1b, b1b, onehot)


def _colbcast(vec_row):
    r = lax.broadcasted_iota(jnp.int32, (C, C), 0)
    q = lax.broadcasted_iota(jnp.int32, (C, C), 1)
    eye = jnp.where(r == q, 1.0, 0.0).astype(jnp.float32)
    colv = lax.dot_general(eye, vec_row, (((1,), (1,)), ((), ())),
                           preferred_element_type=jnp.float32)
    return jnp.broadcast_to(colv, (C, C))


def _expsum_body(s_ref, oh_ref, m_ref, ex_ref, ss_ref):
    i = pl.program_id(0)
    mrow = m_ref[0:1, :]
    mrow = jnp.where(mrow > NEG_BIG, mrow, 0.0)
    mb = jnp.dot(oh_ref[...], _colbcast(mrow),
                 preferred_element_type=jnp.float32)
    ex = jnp.exp(s_ref[...] - mb)
    ex_ref[...] = ex
    contrib = jnp.sum(oh_ref[...] * ex, axis=0, keepdims=True)

    @pl.when(i == 0)
    def _():
        ss_ref[...] = jnp.zeros((8, C), jnp.float32)

    ss_ref[0:1, :] = ss_ref[0:1, :] + contrib


def _expsum(s128, onehot, m128):
    return pl.pallas_call(
        _expsum_body,
        grid=(NBLK,),
        in_specs=[_rowspec(), _rowspec(), _accspec()],
        out_specs=[_rowspec(), _accspec()],
        out_shape=[jax.ShapeDtypeStruct((NPAD, C), jnp.float32),
                   jax.ShapeDtypeStruct((8, C), jnp.float32)],
    )(s128, onehot, m128)


def _norm_body(ex_ref, oh_ref, ss_ref, o_ref):
    ssrow = ss_ref[0:1, :]
    ssrow = jnp.where(ssrow > 0, ssrow, 1.0)
    ssb = jnp.dot(oh_ref[...], _colbcast(ssrow),
                  preferred_element_type=jnp.float32)
    ssb = jnp.where(ssb > 0, ssb, 1.0)
    o_ref[...] = ex_ref[...] / ssb


def _norm(ex128, onehot, ss128):
    return pl.pallas_call(
        _norm_body,
        grid=(NBLK,),
        in_specs=[_rowspec(), _rowspec(), _accspec()],
        out_specs=_rowspec(),
        out_shape=jax.ShapeDtypeStruct((NPAD, C), jnp.float32),
    )(ex128, onehot, ss128)


def kernel(x, edge_index, batch, conv_W0, conv_b0, conv_W1, conv_b1,
           conv_W2, conv_b2, conv_W3, conv_b3, ro_W0, ro_b0, ro_W1, ro_b1):
    row = edge_index[0]
    col = edge_index[1]

    blocks = jnp.arange(1, 8, dtype=jnp.int32) * BLK
    order_c = jnp.argsort(col)
    colc = col[order_c]
    rowe = jnp.concatenate([row[order_c],
                            jnp.zeros((EPAD,), jnp.int32)])
    loce = jnp.concatenate([colc - (colc // BLK) * BLK,
                            jnp.full((EPAD,), DUMV, jnp.int32)])
    co = jnp.searchsorted(colc, blocks).astype(jnp.int32)

    order_r = jnp.argsort(row)
    rowr = row[order_r]
    locr = jnp.concatenate([rowr - (rowr // BLK) * BLK,
                            jnp.full((EPAD,), DUMV, jnp.int32)])
    ro = jnp.searchsorted(rowr, blocks).astype(jnp.int32)

    zero = jnp.zeros((1,), jnp.int32)
    evec = jnp.full((1,), E, jnp.int32)
    meta_c = jnp.concatenate([zero, co, evec,
                              jnp.zeros((6,), jnp.int32)])
    meta_r = jnp.concatenate([zero, ro, evec,
                              jnp.zeros((6,), jnp.int32)])

    x_pad = jnp.pad(x, ((0, NPAD - N), (0, C - x.shape[1])))
    w0p = jnp.pad(conv_W0, ((0, 0), (0, C - conv_W0.shape[1]), (0, 0)))
    onehot = jnp.pad((batch[:, None] == jnp.arange(G, dtype=batch.dtype)
                      ).astype(jnp.float32),
                     ((0, NPAD - N), (0, C - G)))
    rob0 = ro_b0.reshape(1, C)
    w1b = jnp.broadcast_to(ro_W1, (C, C))
    rob1 = jnp.broadcast_to(ro_b1.reshape(1, 1), (1, C))
    biases = (conv_b0.reshape(1, C), conv_b1.reshape(1, C),
              conv_b2.reshape(1, C), conv_b3.reshape(1, C))
    weights = (w0p, conv_W1, conv_W2, conv_W3)

    deg128 = _deg_kernel(locr, meta_r)
    dis128, hs = _prep(deg128, x_pad)

    h = x_pad
    for layer in range(4):
        wk = weights[layer]
        p1 = _spmm_kernel(hs, rowe, loce, meta_c)
        acc, txm1, hs1 = _step1(h, p1, dis128, wk[0], wk[1])
        txm2 = h
        hs_cur = hs1
        for k in range(2, KORD):
            pk = _spmm_kernel(hs_cur, rowe, loce, meta_c)
            if k < KORD - 1:
                acc, tx, hs_cur = _stepm(pk, txm2, acc, dis128, wk[k])
                txm2, txm1 = txm1, tx
            else:
                h, hs = _stepl(pk, txm2, acc, dis128, wk[k], biases[layer])

    s128, m128 = _readout(h, ro_W0, rob0, w1b, rob1, onehot)
    ex128, ss128 = _expsum(s128, onehot, m128)
    out128 = _norm(ex128, onehot, ss128)
    return out128[:N, 0]


# speedup vs baseline: 6.5939x; 1.0072x over previous
import functools

import jax
import jax.numpy as jnp
from jax import lax
from jax.experimental import pallas as pl
from jax.experimental.pallas import tpu as pltpu
from jax.experimental.pallas import tpu_sc as plsc

N = 50000
E = 800000
C = 128
KORD = 6
G = 64

NC, NS, LANES = 2, 16, 16
NW = NC * NS

BLK = 6272
NPAD = 8 * BLK
DUMV = BLK
ACC_ROWS = BLK + 16
CH = 256
EPAD = 1536
BN = 512
NBLK = NPAD // BN
ZR = 98

_mesh = plsc.VectorSubcoreMesh(core_axis_name="c", subcore_axis_name="s")

NEG_BIG = -3.0e38


def _mask_chunk(lidx, e0, lo, hi, nvec):
    iota = lax.iota(jnp.int32, 16)
    for w in range(nvec):
        pos = iota + (e0 + 16 * w)
        lv = lidx[pl.ds(16 * w, 16)]
        keep = jnp.logical_and(pos >= lo, pos < hi)
        lidx[pl.ds(16 * w, 16)] = jnp.where(keep, lv, DUMV)


def _extract9(w, b):
    lo = w[0]
    hi = w[1]
    for i in range(1, 8):
        sel = b >= i
        lo = jnp.where(sel, w[i], lo)
        hi = jnp.where(sel, w[i + 1], hi)
    return lo, hi


@functools.partial(
    pl.kernel,
    out_type=jax.ShapeDtypeStruct((NPAD, C), jnp.float32),
    mesh=_mesh,
    scratch_types=[
        pltpu.VMEM((16,), jnp.int32),
        pltpu.VMEM((CH,), jnp.int32),
        pltpu.VMEM((CH,), jnp.int32),
        pltpu.VMEM((CH,), jnp.int32),
        pltpu.VMEM((CH,), jnp.int32),
        pltpu.VMEM((CH, C), jnp.float32),
        pltpu.VMEM((CH, C), jnp.float32),
        pltpu.VMEM((ZR, C), jnp.float32),
        pltpu.VMEM_SHARED((ACC_ROWS, C), jnp.float32),
        pltpu.SemaphoreType.DMA,
        pltpu.SemaphoreType.DMA,
        pltpu.SemaphoreType.DMA,
        pltpu.SemaphoreType.DMA,
        pltpu.SemaphoreType.DMA,
        pltpu.SemaphoreType.DMA,
        pltpu.SemaphoreType.DMA,
        pltpu.SemaphoreType.DMA,
    ],
)
def _spmm_kernel(hs, rowe, loce, meta, out,
                 mbuf, ridxa, ridxb, lidxa, lidxb, stga, stgb, zbuf, acc,
                 sga, sgb, ssa, ssb, sia, sja, sib, sjb):
    c = lax.axis_index("c")
    s = lax.axis_index("s")
    pltpu.sync_copy(meta.at[pl.ds(0, 16)], mbuf)
    w = mbuf[pl.ds(0, 16)]
    for u in range(ZR):
        for q in range(C // 16):
            zbuf[u, pl.ds(16 * q, 16)] = jnp.zeros((16,), jnp.float32)
    for u in range(4):
        b = 4 * c + u
        lo, hi = _extract9(w, b)
        base_node = b * BLK
        row0 = s * (BLK // NS)
        for z in range(BLK // NS // ZR):
            pltpu.sync_copy(zbuf, acc.at[pl.ds(row0 + z * ZR, ZR)])
        plsc.subcore_barrier()

        span = hi - lo
        share = lax.shift_left(
            lax.shift_right_logical(span + (NS * CH - 1), 12), 8)
        a = lo + s * share
        bnd = jnp.minimum(a + share, hi)
        a8 = jnp.bitwise_and(a, jnp.int32(-8))
        npair = lax.shift_right_logical(
            jnp.maximum(bnd - a8, 0) + (2 * CH - 1), 9)

        def pair_body(j, carry):
            e0 = pl.multiple_of(a8 + j * (2 * CH), 8)
            e1 = pl.multiple_of(e0 + CH, 8)

            @pl.when(j > 0)
            def _():
                pltpu.make_async_copy(stga, acc.at[lidxa], ssa).wait()
                pltpu.make_async_copy(stgb, acc.at[lidxb], ssb).wait()

            ira = pltpu.async_copy(rowe.at[pl.ds(e0, CH)], ridxa, sia)
            ila = pltpu.async_copy(loce.at[pl.ds(e0, CH)], lidxa, sja)
            irb = pltpu.async_copy(rowe.at[pl.ds(e1, CH)], ridxb, sib)
            ilb = pltpu.async_copy(loce.at[pl.ds(e1, CH)], lidxb, sjb)
            ira.wait()
            ila.wait()
            _mask_chunk(lidxa, e0, a, bnd, CH // 16)
            cpa = pltpu.async_copy(hs.at[ridxa], stga, sga)
            irb.wait()
            ilb.wait()
            _mask_chunk(lidxb, e1, a, bnd, CH // 16)
            cpb = pltpu.async_copy(hs.at[ridxb], stgb, sgb)
            cpa.wait()
            pltpu.async_copy(stga, acc.at[lidxa], ssa, add=True)
            cpb.wait()
            pltpu.async_copy(stgb, acc.at[lidxb], ssb, add=True)
            return carry

        lax.fori_loop(0, npair, pair_body, jnp.int32(0))

        @pl.when(npair > 0)
        def _():
            pltpu.make_async_copy(stga, acc.at[lidxa], ssa).wait()
            pltpu.make_async_copy(stgb, acc.at[lidxb], ssb).wait()
        plsc.subcore_barrier()
        nwb = BLK // NS
        pltpu.sync_copy(acc.at[pl.ds(row0, nwb)],
                        out.at[pl.ds(base_node + row0, nwb)])
        plsc.subcore_barrier()


@functools.partial(
    pl.kernel,
    out_type=jax.ShapeDtypeStruct((NPAD, C), jnp.float32),
    mesh=_mesh,
    scratch_types=[
        pltpu.VMEM((16,), jnp.int32),
        pltpu.VMEM((CH,), jnp.int32),
        pltpu.VMEM((CH,), jnp.int32),
        pltpu.VMEM((CH, C), jnp.float32),
        pltpu.VMEM((ZR, C), jnp.float32),
        pltpu.VMEM_SHARED((ACC_ROWS, C), jnp.float32),
        pltpu.SemaphoreType.DMA,
        pltpu.SemaphoreType.DMA,
        pltpu.SemaphoreType.DMA,
        pltpu.SemaphoreType.DMA,
    ],
)
def _deg_kernel(locr, meta, out, mbuf, lidxa, lidxb, ones, zbuf, acc,
                ssa, ssb, sia, sib):
    c = lax.axis_index("c")
    s = lax.axis_index("s")
    pltpu.sync_copy(meta.at[pl.ds(0, 16)], mbuf)
    w = mbuf[pl.ds(0, 16)]
    for u in range(ZR):
        for q in range(C // 16):
            zbuf[u, pl.ds(16 * q, 16)] = jnp.zeros((16,), jnp.float32)
    for u in range(CH):
        for q in range(C // 16):
            ones[u, pl.ds(16 * q, 16)] = jnp.full((16,), 1.0, jnp.float32)
    for u in range(4):
        b = 4 * c + u
        lo, hi = _extract9(w, b)
        base_node = b * BLK
        row0 = s * (BLK // NS)
        for z in range(BLK // NS // ZR):
            pltpu.sync_copy(zbuf, acc.at[pl.ds(row0 + z * ZR, ZR)])
        plsc.subcore_barrier()

        span = hi - lo
        share = lax.shift_left(
            lax.shift_right_logical(span + (NS * CH - 1), 12), 8)
        a = lo + s * share
        bnd = jnp.minimum(a + share, hi)
        a8 = jnp.bitwise_and(a, jnp.int32(-8))
        npair = lax.shift_right_logical(
            jnp.maximum(bnd - a8, 0) + (2 * CH - 1), 9)

        def pair_body(j, carry):
            e0 = pl.multiple_of(a8 + j * (2 * CH), 8)
            e1 = pl.multiple_of(e0 + CH, 8)

            @pl.when(j > 0)
            def _():
                pltpu.make_async_copy(ones, acc.at[lidxa], ssa).wait()
                pltpu.make_async_copy(ones, acc.at[lidxb], ssb).wait()

            ila = pltpu.async_copy(locr.at[pl.ds(e0, CH)], lidxa, sia)
            ilb = pltpu.async_copy(locr.at[pl.ds(e1, CH)], lidxb, sib)
            ila.wait()
            _mask_chunk(lidxa, e0, a, bnd, CH // 16)
            pltpu.async_copy(ones, acc.at[lidxa], ssa, add=True)
            ilb.wait()
            _mask_chunk(lidxb, e1, a, bnd, CH // 16)
            pltpu.async_copy(ones, acc.at[lidxb], ssb, add=True)
            return carry

        lax.fori_loop(0, npair, pair_body, jnp.int32(0))

        @pl.when(npair > 0)
        def _():
            pltpu.make_async_copy(ones, acc.at[lidxa], ssa).wait()
            pltpu.make_async_copy(ones, acc.at[lidxb], ssb).wait()
        plsc.subcore_barrier()
        nwb = BLK // NS
        pltpu.sync_copy(acc.at[pl.ds(row0, nwb)],
                        out.at[pl.ds(base_node + row0, nwb)])
        plsc.subcore_barrier()


def _rowspec():
    return pl.BlockSpec((BN, C), lambda i: (i, 0))


def _wspec():
    return pl.BlockSpec((C, C), lambda i: (0, 0))


def _bspec():
    return pl.BlockSpec((1, C), lambda i: (0, 0))


def _accspec():
    return pl.BlockSpec((8, C), lambda i: (0, 0))


def _prep_body(deg_ref, x_ref, dis_ref, hs_ref):
    d = deg_ref[...]
    safe = jnp.where(d > 0, d, 1.0)
    dis = jnp.where(d > 0, 1.0 / jnp.sqrt(safe), 0.0)
    dis_ref[...] = dis
    hs_ref[...] = dis * x_ref[...]


def _prep(deg128, x_pad):
    return pl.pallas_call(
        _prep_body,
        grid=(NBLK,),
        in_specs=[_rowspec(), _rowspec()],
        out_specs=[_rowspec(), _rowspec()],
        out_shape=[jax.ShapeDtypeStruct((NPAD, C), jnp.float32),
                   jax.ShapeDtypeStruct((NPAD, C), jnp.float32)],
    )(deg128, x_pad)


def _el1_body(p_ref, dis_ref, tx_ref, hs_ref):
    dis = dis_ref[...]
    t1 = -dis * p_ref[...]
    tx_ref[...] = t1
    hs_ref[...] = dis * t1


def _el1(p, dis128):
    return pl.pallas_call(
        _el1_body,
        grid=(NBLK,),
        in_specs=[_rowspec(), _rowspec()],
        out_specs=[_rowspec(), _rowspec()],
        out_shape=[jax.ShapeDtypeStruct((NPAD, C), jnp.float32)] * 2,
    )(p, dis128)


def _elm_body(p_ref, txm2_ref, dis_ref, tx_ref, hs_ref):
    dis = dis_ref[...]
    t = -2.0 * dis * p_ref[...] - txm2_ref[...]
    tx_ref[...] = t
    hs_ref[...] = dis * t


def _elm(p, txm2, dis128):
    return pl.pallas_call(
        _elm_body,
        grid=(NBLK,),
        in_specs=[_rowspec(), _rowspec(), _rowspec()],
        out_specs=[_rowspec(), _rowspec()],
        out_shape=[jax.ShapeDtypeStruct((NPAD, C), jnp.float32)] * 2,
    )(p, txm2, dis128)


def _mm1_body(h_ref, tx_ref, w0_ref, w1_ref, acc_ref):
    acc = jnp.dot(h_ref[...], w0_ref[...], preferred_element_type=jnp.float32)
    acc = acc + jnp.dot(tx_ref[...], w1_ref[...],
                        preferred_element_type=jnp.float32)
    acc_ref[...] = acc


def _mm1(h, tx1, w0, w1):
    return pl.pallas_call(
        _mm1_body,
        grid=(NBLK,),
        in_specs=[_rowspec(), _rowspec(), _wspec(), _wspec()],
        out_specs=_rowspec(),
        out_shape=jax.ShapeDtypeStruct((NPAD, C), jnp.float32),
    )(h, tx1, w0, w1)


def _mmk_body(acc_in_ref, tx_ref, w_ref, acc_ref):
    acc_ref[...] = acc_in_ref[...] + jnp.dot(
        tx_ref[...], w_ref[...], preferred_element_type=jnp.float32)


def _mmk(acc, tx, wk):
    return pl.pallas_call(
        _mmk_body,
        grid=(NBLK,),
        in_specs=[_rowspec(), _rowspec(), _wspec()],
        out_specs=_rowspec(),
        out_shape=jax.ShapeDtypeStruct((NPAD, C), jnp.float32),
    )(acc, tx, wk)


def _fin_body(acc_ref, tx_ref, w_ref, b_ref, dis_ref, h_ref, hs_ref):
    acc = acc_ref[...] + jnp.dot(tx_ref[...], w_ref[...],
                                 preferred_element_type=jnp.float32)
    h = jnp.maximum(acc + b_ref[...], 0.0)
    h_ref[...] = h
    hs_ref[...] = dis_ref[...] * h


def _fin(acc, tx, wk, bias, dis128):
    return pl.pallas_call(
        _fin_body,
        grid=(NBLK,),
        in_specs=[_rowspec(), _rowspec(), _wspec(), _bspec(), _rowspec()],
        out_specs=[_rowspec(), _rowspec()],
        out_shape=[jax.ShapeDtypeStruct((NPAD, C), jnp.float32)] * 2,
    )(acc, tx, wk, bias, dis128)


def _readout_body(h_ref, w0_ref, b0_ref, w1_ref, b1_ref, oh_ref,
                  s_ref, m_ref):
    i = pl.program_id(0)
    z = jnp.maximum(jnp.dot(h_ref[...], w0_ref[...],
                            preferred_element_type=jnp.float32) + b0_ref[...],
                    0.0)
    sv = jnp.maximum(jnp.dot(z, w1_ref[...],
                             preferred_element_type=jnp.float32) + b1_ref[...],
                     0.0)
    s_ref[...] = sv
    oh = oh_ref[...]
    masked = jnp.where(oh > 0.5, sv, NEG_BIG)
    mx = jnp.max(masked, axis=0, keepdims=True)

    @pl.when(i == 0)
    def _():
        m_ref[...] = jnp.full((8, C), NEG_BIG, jnp.float32)

    m_ref[0:1, :] = jnp.maximum(m_ref[0:1, :], mx)


def _readout(h, w0, b0, w1b, b1b, onehot):
    return pl.pallas_call(
        _readout_body,
        grid=(NBLK,),
        in_specs=[_rowspec(), _wspec(), _bspec(), _wspec(), _bspec(),
                  _rowspec()],
        out_specs=[_rowspec(), _accspec()],
        out_shape=[jax.ShapeDtypeStruct((NPAD, C), jnp.float32),
                   jax.ShapeDtypeStruct((8, C), jnp.float32)],
    )(h, w0, b0, w1b, b1b, onehot)


def _colbcast(vec_row):
    r = lax.broadcasted_iota(jnp.int32, (C, C), 0)
    q = lax.broadcasted_iota(jnp.int32, (C, C), 1)
    eye = jnp.where(r == q, 1.0, 0.0).astype(jnp.float32)
    colv = lax.dot_general(eye, vec_row, (((1,), (1,)), ((), ())),
                           preferred_element_type=jnp.float32)
    return jnp.broadcast_to(colv, (C, C))


def _expsum_body(s_ref, oh_ref, m_ref, ex_ref, ss_ref):
    i = pl.program_id(0)
    mrow = m_ref[0:1, :]
    mrow = jnp.where(mrow > NEG_BIG, mrow, 0.0)
    mb = jnp.dot(oh_ref[...], _colbcast(mrow),
                 preferred_element_type=jnp.float32)
    ex = jnp.exp(s_ref[...] - mb)
    ex_ref[...] = ex
    contrib = jnp.sum(oh_ref[...] * ex, axis=0, keepdims=True)

    @pl.when(i == 0)
    def _():
        ss_ref[...] = jnp.zeros((8, C), jnp.float32)

    ss_ref[0:1, :] = ss_ref[0:1, :] + contrib


def _expsum(s128, onehot, m128):
    return pl.pallas_call(
        _expsum_body,
        grid=(NBLK,),
        in_specs=[_rowspec(), _rowspec(), _accspec()],
        out_specs=[_rowspec(), _accspec()],
        out_shape=[jax.ShapeDtypeStruct((NPAD, C), jnp.float32),
                   jax.ShapeDtypeStruct((8, C), jnp.float32)],
    )(s128, onehot, m128)


def _norm_body(ex_ref, oh_ref, ss_ref, o_ref):
    ssrow = ss_ref[0:1, :]
    ssrow = jnp.where(ssrow > 0, ssrow, 1.0)
    ssb = jnp.dot(oh_ref[...], _colbcast(ssrow),
                  preferred_element_type=jnp.float32)
    ssb = jnp.where(ssb > 0, ssb, 1.0)
    o_ref[...] = ex_ref[...] / ssb


def _norm(ex128, onehot, ss128):
    return pl.pallas_call(
        _norm_body,
        grid=(NBLK,),
        in_specs=[_rowspec(), _rowspec(), _accspec()],
        out_specs=_rowspec(),
        out_shape=jax.ShapeDtypeStruct((NPAD, C), jnp.float32),
    )(ex128, onehot, ss128)


def kernel(x, edge_index, batch, conv_W0, conv_b0, conv_W1, conv_b1,
           conv_W2, conv_b2, conv_W3, conv_b3, ro_W0, ro_b0, ro_W1, ro_b1):
    row = edge_index[0]
    col = edge_index[1]

    blocks = jnp.arange(1, 8, dtype=jnp.int32) * BLK
    order_c = jnp.argsort(col)
    colc = col[order_c]
    rowe = jnp.concatenate([row[order_c],
                            jnp.zeros((EPAD,), jnp.int32)])
    loce = jnp.concatenate([colc - (colc // BLK) * BLK,
                            jnp.full((EPAD,), DUMV, jnp.int32)])
    co = jnp.searchsorted(colc, blocks).astype(jnp.int32)

    order_r = jnp.argsort(row)
    rowr = row[order_r]
    locr = jnp.concatenate([rowr - (rowr // BLK) * BLK,
                            jnp.full((EPAD,), DUMV, jnp.int32)])
    ro = jnp.searchsorted(rowr, blocks).astype(jnp.int32)

    zero = jnp.zeros((1,), jnp.int32)
    evec = jnp.full((1,), E, jnp.int32)
    meta_c = jnp.concatenate([zero, co, evec,
                              jnp.zeros((6,), jnp.int32)])
    meta_r = jnp.concatenate([zero, ro, evec,
                              jnp.zeros((6,), jnp.int32)])

    x_pad = jnp.pad(x, ((0, NPAD - N), (0, C - x.shape[1])))
    w0p = jnp.pad(conv_W0, ((0, 0), (0, C - conv_W0.shape[1]), (0, 0)))
    onehot = jnp.pad((batch[:, None] == jnp.arange(G, dtype=batch.dtype)
                      ).astype(jnp.float32),
                     ((0, NPAD - N), (0, C - G)))
    rob0 = ro_b0.reshape(1, C)
    w1b = jnp.broadcast_to(ro_W1, (C, C))
    rob1 = jnp.broadcast_to(ro_b1.reshape(1, 1), (1, C))
    biases = (conv_b0.reshape(1, C), conv_b1.reshape(1, C),
              conv_b2.reshape(1, C), conv_b3.reshape(1, C))
    weights = (w0p, conv_W1, conv_W2, conv_W3)

    deg128 = _deg_kernel(locr, meta_r)
    dis128, hs = _prep(deg128, x_pad)

    h = x_pad
    for layer in range(4):
        wk = weights[layer]
        p = _spmm_kernel(hs, rowe, loce, meta_c)
        txm1, hs_cur = _el1(p, dis128)
        txm2 = h
        acc = None
        for k in range(2, KORD):
            p = _spmm_kernel(hs_cur, rowe, loce, meta_c)
            if k == 2:
                acc = _mm1(h, txm1, wk[0], wk[1])
            else:
                acc = _mmk(acc, txm1, wk[k - 1])
            if k < KORD - 1:
                tx, hs_cur = _elm(p, txm2, dis128)
                txm2, txm1 = txm1, tx
            else:
                tx, _unused = _elm(p, txm2, dis128)
                h, hs = _fin(acc, tx, wk[k], biases[layer], dis128)

    s128, m128 = _readout(h, ro_W0, rob0, w1b, rob1, onehot)
    ex128, ss128 = _expsum(s128, onehot, m128)
    out128 = _norm(ex128, onehot, ss128)
    return out128[:N, 0]


# speedup vs baseline: 6.6749x; 1.0123x over previous
import functools

import jax
import jax.numpy as jnp
from jax import lax
from jax.experimental import pallas as pl
from jax.experimental.pallas import tpu as pltpu
from jax.experimental.pallas import tpu_sc as plsc

N = 50000
E = 800000
C = 128
KORD = 6
G = 64

NC, NS, LANES = 2, 16, 16
NW = NC * NS

BLK = 6272
NPAD = 8 * BLK
DUMV = BLK
ACC_ROWS = BLK + 16
CH = 256
EPAD = 1536
BN = 512
NBLK = NPAD // BN
ZR = 98

_mesh = plsc.VectorSubcoreMesh(core_axis_name="c", subcore_axis_name="s")

NEG_BIG = -3.0e38


def _mask_chunk(lidx, e0, lo, hi, nvec):
    iota = lax.iota(jnp.int32, 16)
    for w in range(nvec):
        pos = iota + (e0 + 16 * w)
        lv = lidx[pl.ds(16 * w, 16)]
        keep = jnp.logical_and(pos >= lo, pos < hi)
        lidx[pl.ds(16 * w, 16)] = jnp.where(keep, lv, DUMV)


def _extract9(w, b):
    lo = w[0]
    hi = w[1]
    for i in range(1, 8):
        sel = b >= i
        lo = jnp.where(sel, w[i], lo)
        hi = jnp.where(sel, w[i + 1], hi)
    return lo, hi


@functools.partial(
    pl.kernel,
    out_type=jax.ShapeDtypeStruct((NPAD, C), jnp.float32),
    mesh=_mesh,
    scratch_types=[
        pltpu.VMEM((16,), jnp.int32),
        pltpu.VMEM((CH,), jnp.int32),
        pltpu.VMEM((CH,), jnp.int32),
        pltpu.VMEM((CH,), jnp.int32),
        pltpu.VMEM((CH,), jnp.int32),
        pltpu.VMEM((CH, C), jnp.float32),
        pltpu.VMEM((CH, C), jnp.float32),
        pltpu.VMEM((ZR, C), jnp.float32),
        pltpu.VMEM_SHARED((ACC_ROWS, C), jnp.float32),
        pltpu.SemaphoreType.DMA,
        pltpu.SemaphoreType.DMA,
        pltpu.SemaphoreType.DMA,
        pltpu.SemaphoreType.DMA,
        pltpu.SemaphoreType.DMA,
        pltpu.SemaphoreType.DMA,
        pltpu.SemaphoreType.DMA,
        pltpu.SemaphoreType.DMA,
    ],
)
def _spmm_kernel(hs, rowe, loce, meta, out,
                 mbuf, ridxa, ridxb, lidxa, lidxb, stga, stgb, zbuf, acc,
                 sga, sgb, ssa, ssb, sia, sja, sib, sjb):
    c = lax.axis_index("c")
    s = lax.axis_index("s")
    pltpu.sync_copy(meta.at[pl.ds(0, 16)], mbuf)
    w = mbuf[pl.ds(0, 16)]
    for u in range(ZR):
        for q in range(C // 16):
            zbuf[u, pl.ds(16 * q, 16)] = jnp.zeros((16,), jnp.float32)
    for u in range(4):
        b = 4 * c + u
        lo, hi = _extract9(w, b)
        base_node = b * BLK
        row0 = s * (BLK // NS)
        for z in range(BLK // NS // ZR):
            pltpu.sync_copy(zbuf, acc.at[pl.ds(row0 + z * ZR, ZR)])
        plsc.subcore_barrier()

        span = hi - lo
        share = lax.shift_left(
            lax.shift_right_logical(span + (NS * CH - 1), 12), 8)
        a = lo + s * share
        bnd = jnp.minimum(a + share, hi)
        a8 = jnp.bitwise_and(a, jnp.int32(-8))
        npair = lax.shift_right_logical(
            jnp.maximum(bnd - a8, 0) + (2 * CH - 1), 9)

        def pair_body(j, carry):
            e0 = pl.multiple_of(a8 + j * (2 * CH), 8)
            e1 = pl.multiple_of(e0 + CH, 8)

            @pl.when(j > 0)
            def _():
                pltpu.make_async_copy(stga, acc.at[lidxa], ssa).wait()
                pltpu.make_async_copy(stgb, acc.at[lidxb], ssb).wait()

            ira = pltpu.async_copy(rowe.at[pl.ds(e0, CH)], ridxa, sia)
            ila = pltpu.async_copy(loce.at[pl.ds(e0, CH)], lidxa, sja)
            irb = pltpu.async_copy(rowe.at[pl.ds(e1, CH)], ridxb, sib)
            ilb = pltpu.async_copy(loce.at[pl.ds(e1, CH)], lidxb, sjb)
            ira.wait()
            ila.wait()
            _mask_chunk(lidxa, e0, a, bnd, CH // 16)
            cpa = pltpu.async_copy(hs.at[ridxa], stga, sga)
            irb.wait()
            ilb.wait()
            _mask_chunk(lidxb, e1, a, bnd, CH // 16)
            cpb = pltpu.async_copy(hs.at[ridxb], stgb, sgb)
            cpa.wait()
            pltpu.async_copy(stga, acc.at[lidxa], ssa, add=True)
            cpb.wait()
            pltpu.async_copy(stgb, acc.at[lidxb], ssb, add=True)
            return carry

        lax.fori_loop(0, npair, pair_body, jnp.int32(0))

        @pl.when(npair > 0)
        def _():
            pltpu.make_async_copy(stga, acc.at[lidxa], ssa).wait()
            pltpu.make_async_copy(stgb, acc.at[lidxb], ssb).wait()
        plsc.subcore_barrier()
        nwb = BLK // NS
        pltpu.sync_copy(acc.at[pl.ds(row0, nwb)],
                        out.at[pl.ds(base_node + row0, nwb)])
        plsc.subcore_barrier()


@functools.partial(
    pl.kernel,
    out_type=jax.ShapeDtypeStruct((NPAD, C), jnp.float32),
    mesh=_mesh,
    scratch_types=[
        pltpu.VMEM((16,), jnp.int32),
        pltpu.VMEM((CH,), jnp.int32),
        pltpu.VMEM((CH,), jnp.int32),
        pltpu.VMEM((CH, C), jnp.float32),
        pltpu.VMEM((ZR, C), jnp.float32),
        pltpu.VMEM_SHARED((ACC_ROWS, C), jnp.float32),
        pltpu.SemaphoreType.DMA,
        pltpu.SemaphoreType.DMA,
        pltpu.SemaphoreType.DMA,
        pltpu.SemaphoreType.DMA,
    ],
)
def _deg_kernel(locr, meta, out, mbuf, lidxa, lidxb, ones, zbuf, acc,
                ssa, ssb, sia, sib):
    c = lax.axis_index("c")
    s = lax.axis_index("s")
    pltpu.sync_copy(meta.at[pl.ds(0, 16)], mbuf)
    w = mbuf[pl.ds(0, 16)]
    for u in range(ZR):
        for q in range(C // 16):
            zbuf[u, pl.ds(16 * q, 16)] = jnp.zeros((16,), jnp.float32)
    for u in range(CH):
        for q in range(C // 16):
            ones[u, pl.ds(16 * q, 16)] = jnp.full((16,), 1.0, jnp.float32)
    for u in range(4):
        b = 4 * c + u
        lo, hi = _extract9(w, b)
        base_node = b * BLK
        row0 = s * (BLK // NS)
        for z in range(BLK // NS // ZR):
            pltpu.sync_copy(zbuf, acc.at[pl.ds(row0 + z * ZR, ZR)])
        plsc.subcore_barrier()

        span = hi - lo
        share = lax.shift_left(
            lax.shift_right_logical(span + (NS * CH - 1), 12), 8)
        a = lo + s * share
        bnd = jnp.minimum(a + share, hi)
        a8 = jnp.bitwise_and(a, jnp.int32(-8))
        npair = lax.shift_right_logical(
            jnp.maximum(bnd - a8, 0) + (2 * CH - 1), 9)

        def pair_body(j, carry):
            e0 = pl.multiple_of(a8 + j * (2 * CH), 8)
            e1 = pl.multiple_of(e0 + CH, 8)

            @pl.when(j > 0)
            def _():
                pltpu.make_async_copy(ones, acc.at[lidxa], ssa).wait()
                pltpu.make_async_copy(ones, acc.at[lidxb], ssb).wait()

            ila = pltpu.async_copy(locr.at[pl.ds(e0, CH)], lidxa, sia)
            ilb = pltpu.async_copy(locr.at[pl.ds(e1, CH)], lidxb, sib)
            ila.wait()
            _mask_chunk(lidxa, e0, a, bnd, CH // 16)
            pltpu.async_copy(ones, acc.at[lidxa], ssa, add=True)
            ilb.wait()
            _mask_chunk(lidxb, e1, a, bnd, CH // 16)
            pltpu.async_copy(ones, acc.at[lidxb], ssb, add=True)
            return carry

        lax.fori_loop(0, npair, pair_body, jnp.int32(0))

        @pl.when(npair > 0)
        def _():
            pltpu.make_async_copy(ones, acc.at[lidxa], ssa).wait()
            pltpu.make_async_copy(ones, acc.at[lidxb], ssb).wait()
        plsc.subcore_barrier()
        nwb = BLK // NS
        pltpu.sync_copy(acc.at[pl.ds(row0, nwb)],
                        out.at[pl.ds(base_node + row0, nwb)])
        plsc.subcore_barrier()


def _rowspec():
    return pl.BlockSpec((BN, C), lambda i: (i, 0))


def _wspec():
    return pl.BlockSpec((C, C), lambda i: (0, 0))


def _bspec():
    return pl.BlockSpec((1, C), lambda i: (0, 0))


def _accspec():
    return pl.BlockSpec((8, C), lambda i: (0, 0))


def _prep_body(deg_ref, x_ref, dis_ref, hs_ref):
    d = deg_ref[...]
    safe = jnp.where(d > 0, d, 1.0)
    dis = jnp.where(d > 0, 1.0 / jnp.sqrt(safe), 0.0)
    dis_ref[...] = dis
    hs_ref[...] = dis * x_ref[...]


def _prep(deg128, x_pad):
    return pl.pallas_call(
        _prep_body,
        grid=(NBLK,),
        in_specs=[_rowspec(), _rowspec()],
        out_specs=[_rowspec(), _rowspec()],
        out_shape=[jax.ShapeDtypeStruct((NPAD, C), jnp.float32),
                   jax.ShapeDtypeStruct((NPAD, C), jnp.float32)],
    )(deg128, x_pad)


def _el1_body(p_ref, dis_ref, tx_ref, hs_ref):
    dis = dis_ref[...]
    t1 = -dis * p_ref[...]
    tx_ref[...] = t1
    hs_ref[...] = dis * t1


def _el1(p, dis128):
    return pl.pallas_call(
        _el1_body,
        grid=(NBLK,),
        in_specs=[_rowspec(), _rowspec()],
        out_specs=[_rowspec(), _rowspec()],
        out_shape=[jax.ShapeDtypeStruct((NPAD, C), jnp.float32)] * 2,
    )(p, dis128)


def _elm_body(p_ref, txm2_ref, dis_ref, tx_ref, hs_ref):
    dis = dis_ref[...]
    t = -2.0 * dis * p_ref[...] - txm2_ref[...]
    tx_ref[...] = t
    hs_ref[...] = dis * t


def _elm(p, txm2, dis128):
    return pl.pallas_call(
        _elm_body,
        grid=(NBLK,),
        in_specs=[_rowspec(), _rowspec(), _rowspec()],
        out_specs=[_rowspec(), _rowspec()],
        out_shape=[jax.ShapeDtypeStruct((NPAD, C), jnp.float32)] * 2,
    )(p, txm2, dis128)


def _mm1_body(h_ref, tx_ref, w0_ref, w1_ref, acc_ref):
    acc = jnp.dot(h_ref[...], w0_ref[...], preferred_element_type=jnp.float32)
    acc = acc + jnp.dot(tx_ref[...], w1_ref[...],
                        preferred_element_type=jnp.float32)
    acc_ref[...] = acc


def _mm1(h, tx1, w0, w1):
    return pl.pallas_call(
        _mm1_body,
        grid=(NBLK,),
        in_specs=[_rowspec(), _rowspec(), _wspec(), _wspec()],
        out_specs=_rowspec(),
        out_shape=jax.ShapeDtypeStruct((NPAD, C), jnp.float32),
    )(h, tx1, w0, w1)


def _mmk_body(acc_in_ref, tx_ref, w_ref, acc_ref):
    acc_ref[...] = acc_in_ref[...] + jnp.dot(
        tx_ref[...], w_ref[...], preferred_element_type=jnp.float32)


def _mmk(acc, tx, wk):
    return pl.pallas_call(
        _mmk_body,
        grid=(NBLK,),
        in_specs=[_rowspec(), _rowspec(), _wspec()],
        out_specs=_rowspec(),
        out_shape=jax.ShapeDtypeStruct((NPAD, C), jnp.float32),
    )(acc, tx, wk)


def _fin_body(acc_ref, tx_ref, w_ref, b_ref, dis_ref, h_ref, hs_ref):
    acc = acc_ref[...] + jnp.dot(tx_ref[...], w_ref[...],
                                 preferred_element_type=jnp.float32)
    h = jnp.maximum(acc + b_ref[...], 0.0)
    h_ref[...] = h
    hs_ref[...] = dis_ref[...] * h


def _fin(acc, tx, wk, bias, dis128):
    return pl.pallas_call(
        _fin_body,
        grid=(NBLK,),
        in_specs=[_rowspec(), _rowspec(), _wspec(), _bspec(), _rowspec()],
        out_specs=[_rowspec(), _rowspec()],
        out_shape=[jax.ShapeDtypeStruct((NPAD, C), jnp.float32)] * 2,
    )(acc, tx, wk, bias, dis128)


def _readout_body(h_ref, w0_ref, b0_ref, w1_ref, b1_ref, oh_ref,
                  s_ref, m_ref):
    i = pl.program_id(0)
    z = jnp.maximum(jnp.dot(h_ref[...], w0_ref[...],
                            preferred_element_type=jnp.float32) + b0_ref[...],
                    0.0)
    sv = jnp.maximum(jnp.dot(z, w1_ref[...],
                             preferred_element_type=jnp.float32) + b1_ref[...],
                     0.0)
    s_ref[...] = sv
    oh = oh_ref[...]
    masked = jnp.where(oh > 0.5, sv, NEG_BIG)
    mx = jnp.max(masked, axis=0, keepdims=True)

    @pl.when(i == 0)
    def _():
        m_ref[...] = jnp.full((8, C), NEG_BIG, jnp.float32)

    m_ref[0:1, :] = jnp.maximum(m_ref[0:1, :], mx)


def _readout(h, w0, b0, w1b, b1b, onehot):
    return pl.pallas_call(
        _readout_body,
        grid=(NBLK,),
        in_specs=[_rowspec(), _wspec(), _bspec(), _wspec(), _bspec(),
                  _rowspec()],
        out_specs=[_rowspec(), _accspec()],
        out_shape=[jax.ShapeDtypeStruct((NPAD, C), jnp.float32),
                   jax.ShapeDtypeStruct((8, C), jnp.float32)],
    )(h, w0, b0, w1b, b1b, onehot)


def _colbcast(vec_row):
    r = lax.broadcasted_iota(jnp.int32, (C, C), 0)
    q = lax.broadcasted_iota(jnp.int32, (C, C), 1)
    eye = jnp.where(r == q, 1.0, 0.0).astype(jnp.float32)
    colv = lax.dot_general(eye, vec_row, (((1,), (1,)), ((), ())),
                           preferred_element_type=jnp.float32)
    return jnp.broadcast_to(colv, (C, C))


def _expsum_body(s_ref, oh_ref, m_ref, ex_ref, ss_ref):
    i = pl.program_id(0)
    mrow = m_ref[0:1, :]
    mrow = jnp.where(mrow > NEG_BIG, mrow, 0.0)
    mb = jnp.dot(oh_ref[...], _colbcast(mrow),
                 preferred_element_type=jnp.float32)
    ex = jnp.exp(s_ref[...] - mb)
    ex_ref[...] = ex
    contrib = jnp.sum(oh_ref[...] * ex, axis=0, keepdims=True)

    @pl.when(i == 0)
    def _():
        ss_ref[...] = jnp.zeros((8, C), jnp.float32)

    ss_ref[0:1, :] = ss_ref[0:1, :] + contrib


def _expsum(s128, onehot, m128):
    return pl.pallas_call(
        _expsum_body,
        grid=(NBLK,),
        in_specs=[_rowspec(), _rowspec(), _accspec()],
        out_specs=[_rowspec(), _accspec()],
        out_shape=[jax.ShapeDtypeStruct((NPAD, C), jnp.float32),
                   jax.ShapeDtypeStruct((8, C), jnp.float32)],
    )(s128, onehot, m128)


def _norm_body(ex_ref, oh_ref, ss_ref, o_ref):
    ssrow = ss_ref[0:1, :]
    ssrow = jnp.where(ssrow > 0, ssrow, 1.0)
    ssb = jnp.dot(oh_ref[...], _colbcast(ssrow),
                  preferred_element_type=jnp.float32)
    ssb = jnp.where(ssb > 0, ssb, 1.0)
    o_ref[...] = ex_ref[...] / ssb


def _norm(ex128, onehot, ss128):
    return pl.pallas_call(
        _norm_body,
        grid=(NBLK,),
        in_specs=[_rowspec(), _rowspec(), _accspec()],
        out_specs=_rowspec(),
        out_shape=jax.ShapeDtypeStruct((NPAD, C), jnp.float32),
    )(ex128, onehot, ss128)


def kernel(x, edge_index, batch, conv_W0, conv_b0, conv_W1, conv_b1,
           conv_W2, conv_b2, conv_W3, conv_b3, ro_W0, ro_b0, ro_W1, ro_b1):
    row = edge_index[0]
    col = edge_index[1]

    blocks = jnp.arange(1, 8, dtype=jnp.int32) * BLK
    colc, rowc = lax.sort((col, row), num_keys=1)
    rowe = jnp.concatenate([rowc, jnp.zeros((EPAD,), jnp.int32)])
    loce = jnp.concatenate([colc - (colc // BLK) * BLK,
                            jnp.full((EPAD,), DUMV, jnp.int32)])
    co = jnp.searchsorted(colc, blocks).astype(jnp.int32)

    rowr = lax.sort((row,), num_keys=1)[0]
    locr = jnp.concatenate([rowr - (rowr // BLK) * BLK,
                            jnp.full((EPAD,), DUMV, jnp.int32)])
    ro = jnp.searchsorted(rowr, blocks).astype(jnp.int32)

    zero = jnp.zeros((1,), jnp.int32)
    evec = jnp.full((1,), E, jnp.int32)
    meta_c = jnp.concatenate([zero, co, evec,
                              jnp.zeros((6,), jnp.int32)])
    meta_r = jnp.concatenate([zero, ro, evec,
                              jnp.zeros((6,), jnp.int32)])

    x_pad = jnp.pad(x, ((0, NPAD - N), (0, C - x.shape[1])))
    w0p = jnp.pad(conv_W0, ((0, 0), (0, C - conv_W0.shape[1]), (0, 0)))
    onehot = jnp.pad((batch[:, None] == jnp.arange(G, dtype=batch.dtype)
                      ).astype(jnp.float32),
                     ((0, NPAD - N), (0, C - G)))
    rob0 = ro_b0.reshape(1, C)
    w1b = jnp.broadcast_to(ro_W1, (C, C))
    rob1 = jnp.broadcast_to(ro_b1.reshape(1, 1), (1, C))
    biases = (conv_b0.reshape(1, C), conv_b1.reshape(1, C),
              conv_b2.reshape(1, C), conv_b3.reshape(1, C))
    weights = (w0p, conv_W1, conv_W2, conv_W3)

    deg128 = _deg_kernel(locr, meta_r)
    dis128, hs = _prep(deg128, x_pad)

    h = x_pad
    for layer in range(4):
        wk = weights[layer]
        p = _spmm_kernel(hs, rowe, loce, meta_c)
        txm1, hs_cur = _el1(p, dis128)
        txm2 = h
        acc = None
        for k in range(2, KORD):
            p = _spmm_kernel(hs_cur, rowe, loce, meta_c)
            if k == 2:
                acc = _mm1(h, txm1, wk[0], wk[1])
            else:
                acc = _mmk(acc, txm1, wk[k - 1])
            if k < KORD - 1:
                tx, hs_cur = _elm(p, txm2, dis128)
                txm2, txm1 = txm1, tx
            else:
                tx, _unused = _elm(p, txm2, dis128)
                h, hs = _fin(acc, tx, wk[k], biases[layer], dis128)

    s128, m128 = _readout(h, ro_W0, rob0, w1b, rob1, onehot)
    ex128, ss128 = _expsum(s128, onehot, m128)
    out128 = _norm(ex128, onehot, ss128)
    return out128[:N, 0]
